# Initial kernel scaffold; baseline (speedup 1.0000x reference)
import jax
import jax.numpy as jnp
from jax.experimental import pallas as pl


def kernel(x, edge_index, edge_attr, edge_weight, node_weight, batch, W0, b0, W1, b1, W2, b2, W3, b3, Wf, bf, Wm1, bm1, Wm2, bm2):
    raise NotImplementedError("write your pallas kernel here")


# speedup vs baseline: 1.0564x; 1.0564x over previous
import jax
import jax.numpy as jnp
from jax.experimental import pallas as pl
from jax.experimental.pallas import tpu as pltpu

N_GRAPHS_C = 32


def _readout_kernel(hw_ref, bsel_ref, Wm1_ref, bm1_ref, Wm2_ref, bm2_ref, out_ref, acc):
    i = pl.program_id(0)
    n = pl.num_programs(0)

    @pl.when(i == 0)
    def _():
        acc[...] = jnp.zeros_like(acc)

    bids = bsel_ref[0, 0, :]
    onehot = (jax.lax.broadcasted_iota(jnp.int32, (N_GRAPHS_C, bids.shape[0]), 0)
              == bids[None, :]).astype(jnp.float32)
    acc[:, :-1] += jax.lax.dot_general(
        onehot, hw_ref[...], (((1,), (0,)), ((), ())),
        preferred_element_type=jnp.float32)
    acc[:, -1:] += jnp.sum(onehot, axis=1, keepdims=True)

    @pl.when(i == n - 1)
    def _():
        gsum = acc[:, :-1]
        gcnt = acc[:, -1:]
        ge = gsum / jnp.maximum(gcnt, 1.0)
        hmid = jax.nn.relu(
            jax.lax.dot_general(ge, Wm1_ref[...], (((1,), (0,)), ((), ())),
                                preferred_element_type=jnp.float32) + bm1_ref[...])
        out_ref[...] = (jax.lax.dot_general(hmid, Wm2_ref[...], (((1,), (0,)), ((), ())),
                                            preferred_element_type=jnp.float32)
                        + bm2_ref[...])


def _readout(hw, batch, Wm1, bm1, Wm2, bm2):
    n, d = hw.shape
    BLK = 1000
    grid = n // BLK
    batch2 = batch.reshape(grid, 1, BLK)
    return pl.pallas_call(
        _readout_kernel,
        grid=(grid,),
        in_specs=[
            pl.BlockSpec((BLK, d), lambda i: (i, 0)),
            pl.BlockSpec((1, 1, BLK), lambda i: (i, 0, 0)),
            pl.BlockSpec((d, Wm1.shape[1]), lambda i: (0, 0)),
            pl.BlockSpec((1, Wm1.shape[1]), lambda i: (0, 0)),
            pl.BlockSpec((Wm2.shape[0], Wm2.shape[1]), lambda i: (0, 0)),
            pl.BlockSpec((1, 1), lambda i: (0, 0)),
        ],
        out_specs=pl.BlockSpec((N_GRAPHS_C, 1), lambda i: (0, 0)),
        out_shape=jax.ShapeDtypeStruct((N_GRAPHS_C, 1), jnp.float32),
        scratch_shapes=[pltpu.VMEM((N_GRAPHS_C, d + 1), jnp.float32)],
    )(hw, batch2, Wm1, bm1.reshape(1, -1), Wm2, bm2.reshape(1, 1))


def kernel(x, edge_index, edge_attr, edge_weight, node_weight, batch,
           W0, b0, W1, b1, W2, b2, W3, b3, Wf, bf, Wm1, bm1, Wm2, bm2):
    n_nodes = x.shape[0]
    src = edge_index[0]
    dst = edge_index[1]
    inc = jax.ops.segment_sum(edge_weight[:, None] * edge_attr, dst, num_segments=n_nodes)
    h0 = jax.nn.relu(jnp.concatenate([x, inc], axis=1) @ W0 + b0)
    cnt = jax.ops.segment_sum(jnp.ones_like(edge_weight), dst, num_segments=n_nodes)
    denom = jnp.maximum(cnt, 1.0)[:, None]
    h = h0
    for W, b in ((W1, b1), (W2, b2), (W3, b3)):
        msgs = edge_weight[:, None] * jnp.take(h, src, axis=0)
        s = jax.ops.segment_sum(msgs, dst, num_segments=n_nodes)
        h = jax.nn.relu(h0 + (s / denom) @ W + b)
    hin = jnp.concatenate([h, x], axis=1)
    msgs = edge_weight[:, None] * jnp.take(hin, src, axis=0)
    s = jax.ops.segment_sum(msgs, dst, num_segments=n_nodes)
    h = jax.nn.relu((s / denom) @ Wf + bf)
    hw = h * node_weight[:, None]
    out = _readout(hw, batch, Wm1, bm1, Wm2, bm2)
    return out[:, 0]


# speedup vs baseline: 3.4369x; 3.2535x over previous
import dataclasses
import functools

import jax
import jax.numpy as jnp
from jax import lax
from jax.experimental import pallas as pl
from jax.experimental.pallas import tpu as pltpu
from jax.experimental.pallas import tpu_sc as plsc

N = 10000
E = 320000
P = 320
HD = P // 2
XD = 128
XH = XD // 2
NG = 32
NS = 16
BLK = 1000
HI = jax.lax.Precision.HIGHEST

_mesh = plsc.VectorSubcoreMesh(core_axis_name="c", subcore_axis_name="s")


def _sc_compiler_params():
    cp = pltpu.CompilerParams(use_tc_tiling_on_sc=False)
    if "needs_layout_passes" in pltpu.CompilerParams.__dataclass_fields__:
        cp = dataclasses.replace(cp, needs_layout_passes=False)
    return cp


def _sc_inc_cnt(edge_attr, edge_weight, dst, zeros32):
    K = 400
    EPW = E // (2 * NS)
    NCH = EPW // K

    def body(ea_hbm, ew_hbm, dst_hbm, z_hbm, out_hbm,
             eabuf, ewbuf, dstbuf, rows, acc, sem):
        c = lax.axis_index("c")
        s = lax.axis_index("s")

        @pl.when(s < 10)
        def _():
            pltpu.sync_copy(z_hbm.at[pl.ds(s * 1000, 1000)],
                            acc.at[pl.ds(s * 1000, 1000)])
        plsc.subcore_barrier()

        onevec = jnp.where(lax.iota(jnp.int32, 16) == 0,
                           jnp.float32(1.0), jnp.float32(0.0))
        base = (c * NS + s) * EPW

        @pl.loop(0, NCH)
        def _(i):
            off = base + i * K
            pltpu.sync_copy(ea_hbm.at[pl.ds(off, K)], eabuf)
            pltpu.sync_copy(ew_hbm.at[pl.ds(off, K)], ewbuf)
            pltpu.sync_copy(dst_hbm.at[pl.ds(off, K)], dstbuf)

            @pl.loop(0, K // 16)
            def _(g):
                wv = ewbuf[pl.ds(g * 16, 16)]
                for j in range(16):
                    e = g * 16 + j
                    rows[e, 0:16] = eabuf[e, :] * wv[j]
                    rows[e, 16:32] = onevec

            pltpu.sync_copy(rows, acc.at[dstbuf], add=True)

        plsc.subcore_barrier()

        @pl.when(s < 10)
        def _():
            pltpu.sync_copy(acc.at[pl.ds(s * 1000, 1000)],
                            out_hbm.at[c, pl.ds(s * 1000, 1000)])

    f = pl.kernel(
        body,
        out_type=jax.ShapeDtypeStruct((2, N, 32), jnp.float32),
        mesh=_mesh,
        scratch_types=[
            pltpu.VMEM((K, 16), jnp.float32),
            pltpu.VMEM((K,), jnp.float32),
            pltpu.VMEM((K,), jnp.int32),
            pltpu.VMEM((K, 32), jnp.float32),
            pltpu.VMEM_SHARED((N, 32), jnp.float32),
            pltpu.SemaphoreType.DMA,
        ],
        compiler_params=_sc_compiler_params(),
    )
    return f(edge_attr, edge_weight, dst, zeros32)


def _sc_gather_scatter(ha, hb, src, dst, edge_weight, zeros, D, K):
    EPW = E // NS
    NCH = EPW // K

    def body(ha_hbm, hb_hbm, src_hbm, dst_hbm, ew_hbm, z_hbm,
             oa_hbm, ob_hbm, srcbuf, dstbuf, ewbuf, rows, acc, sem):
        c = lax.axis_index("c")
        s = lax.axis_index("s")

        @pl.when(s < 10)
        def _():
            pltpu.sync_copy(z_hbm.at[pl.ds(s * 1000, 1000)],
                            acc.at[pl.ds(s * 1000, 1000)])
        plsc.subcore_barrier()

        def run(h_hbm):
            base = s * EPW

            @pl.loop(0, NCH)
            def _(i):
                off = base + i * K
                pltpu.sync_copy(src_hbm.at[pl.ds(off, K)], srcbuf)
                pltpu.sync_copy(dst_hbm.at[pl.ds(off, K)], dstbuf)
                pltpu.sync_copy(ew_hbm.at[pl.ds(off, K)], ewbuf)
                pltpu.async_copy(h_hbm.at[srcbuf], rows, sem).wait()

                @pl.loop(0, K // 16)
                def _(g):
                    wv = ewbuf[pl.ds(g * 16, 16)]
                    for j in range(16):
                        e = g * 16 + j
                        w = wv[j]
                        for d in range(D // 16):
                            rows[e, d * 16:(d + 1) * 16] = (
                                rows[e, d * 16:(d + 1) * 16] * w)

                pltpu.sync_copy(rows, acc.at[dstbuf], add=True)

        @pl.when(c == 0)
        def _():
            run(ha_hbm)

        @pl.when(c == 1)
        def _():
            run(hb_hbm)

        plsc.subcore_barrier()

        @pl.when((s < 10) & (c == 0))
        def _():
            pltpu.sync_copy(acc.at[pl.ds(s * 1000, 1000)],
                            oa_hbm.at[pl.ds(s * 1000, 1000)])

        @pl.when((s < 10) & (c == 1))
        def _():
            pltpu.sync_copy(acc.at[pl.ds(s * 1000, 1000)],
                            ob_hbm.at[pl.ds(s * 1000, 1000)])

    f = pl.kernel(
        body,
        out_type=(jax.ShapeDtypeStruct((N, D), jnp.float32),
                  jax.ShapeDtypeStruct((N, D), jnp.float32)),
        mesh=_mesh,
        scratch_types=[
            pltpu.VMEM((K,), jnp.int32),
            pltpu.VMEM((K,), jnp.int32),
            pltpu.VMEM((K,), jnp.float32),
            pltpu.VMEM((K, D), jnp.float32),
            pltpu.VMEM_SHARED((N, D), jnp.float32),
            pltpu.SemaphoreType.DMA,
        ],
        compiler_params=_sc_compiler_params(),
    )
    return f(ha, hb, src, dst, edge_weight, zeros)


def _dot(a, b):
    return lax.dot_general(a, b, (((1,), (0,)), ((), ())),
                           preferred_element_type=jnp.float32, precision=HI)


def _tc1_kernel(x_ref, incp_ref, w0x_ref, w0e_ref, b0_ref,
                h0a_ref, h0b_ref, invd_ref):
    inc = incp_ref[0] + incp_ref[1]
    h = jnp.maximum(_dot(x_ref[...], w0x_ref[...])
                    + _dot(inc[:, :16], w0e_ref[...]) + b0_ref[...], 0.0)
    h0a_ref[...] = h[:, :HD]
    h0b_ref[...] = h[:, HD:]
    invd_ref[...] = 1.0 / jnp.maximum(inc[:, 16:17], 1.0)


def _tc1(x, incp, W0x, W0e, b0):
    grid = N // BLK
    return pl.pallas_call(
        _tc1_kernel,
        grid=(grid,),
        in_specs=[
            pl.BlockSpec((BLK, XD), lambda i: (i, 0)),
            pl.BlockSpec((2, BLK, 32), lambda i: (0, i, 0)),
            pl.BlockSpec((XD, P), lambda i: (0, 0)),
            pl.BlockSpec((16, P), lambda i: (0, 0)),
            pl.BlockSpec((1, P), lambda i: (0, 0)),
        ],
        out_specs=(
            pl.BlockSpec((BLK, HD), lambda i: (i, 0)),
            pl.BlockSpec((BLK, HD), lambda i: (i, 0)),
            pl.BlockSpec((BLK, 1), lambda i: (i, 0)),
        ),
        out_shape=(
            jax.ShapeDtypeStruct((N, HD), jnp.float32),
            jax.ShapeDtypeStruct((N, HD), jnp.float32),
            jax.ShapeDtypeStruct((N, 1), jnp.float32),
        ),
    )(x, incp, W0x, W0e, b0)


def _tc2_kernel(sa_ref, sb_ref, invd_ref, h0a_ref, h0b_ref, w_ref, b_ref,
                ha_ref, hb_ref):
    aggr = jnp.concatenate([sa_ref[...], sb_ref[...]], axis=1) * invd_ref[...]
    h0 = jnp.concatenate([h0a_ref[...], h0b_ref[...]], axis=1)
    h = jnp.maximum(h0 + _dot(aggr, w_ref[...]) + b_ref[...], 0.0)
    ha_ref[...] = h[:, :HD]
    hb_ref[...] = h[:, HD:]


def _tc2(sa, sb, invd, h0a, h0b, W, b):
    grid = N // BLK
    return pl.pallas_call(
        _tc2_kernel,
        grid=(grid,),
        in_specs=[
            pl.BlockSpec((BLK, HD), lambda i: (i, 0)),
            pl.BlockSpec((BLK, HD), lambda i: (i, 0)),
            pl.BlockSpec((BLK, 1), lambda i: (i, 0)),
            pl.BlockSpec((BLK, HD), lambda i: (i, 0)),
            pl.BlockSpec((BLK, HD), lambda i: (i, 0)),
            pl.BlockSpec((P, P), lambda i: (0, 0)),
            pl.BlockSpec((1, P), lambda i: (0, 0)),
        ],
        out_specs=(
            pl.BlockSpec((BLK, HD), lambda i: (i, 0)),
            pl.BlockSpec((BLK, HD), lambda i: (i, 0)),
        ),
        out_shape=(
            jax.ShapeDtypeStruct((N, HD), jnp.float32),
            jax.ShapeDtypeStruct((N, HD), jnp.float32),
        ),
    )(sa, sb, invd, h0a, h0b, W, b)


def _tc3_kernel(sa_ref, sb_ref, axa_ref, axb_ref, invd_ref, nw_ref, bt_ref,
                wfh_ref, wfx_ref, bf_ref, wm1_ref, bm1_ref, wm2_ref, bm2_ref,
                out_ref, acc):
    i = pl.program_id(0)
    n = pl.num_programs(0)

    @pl.when(i == 0)
    def _():
        acc[...] = jnp.zeros_like(acc)

    invd = invd_ref[...]
    aggh = jnp.concatenate([sa_ref[...], sb_ref[...]], axis=1) * invd
    aggx = jnp.concatenate([axa_ref[...], axb_ref[...]], axis=1) * invd
    h = jnp.maximum(_dot(aggh, wfh_ref[...]) + _dot(aggx, wfx_ref[...])
                    + bf_ref[...], 0.0)
    hw = h * nw_ref[...]
    bids = bt_ref[0, 0, :]
    onehot = (jax.lax.broadcasted_iota(jnp.int32, (NG, BLK), 0)
              == bids[None, :]).astype(jnp.float32)
    acc[:, :P] += _dot(onehot, hw)
    acc[:, P:] += jnp.sum(onehot, axis=1, keepdims=True)

    @pl.when(i == n - 1)
    def _():
        ge = acc[:, :P] / jnp.maximum(acc[:, P:], 1.0)
        hm = jnp.maximum(_dot(ge, wm1_ref[...]) + bm1_ref[...], 0.0)
        out_ref[...] = _dot(hm, wm2_ref[...]) + bm2_ref[...]


def _tc3(sa, sb, axa, axb, invd, nw, batch3, Wfh, Wfx, bf, Wm1, bm1, Wm2, bm2):
    grid = N // BLK
    return pl.pallas_call(
        _tc3_kernel,
        grid=(grid,),
        in_specs=[
            pl.BlockSpec((BLK, HD), lambda i: (i, 0)),
            pl.BlockSpec((BLK, HD), lambda i: (i, 0)),
            pl.BlockSpec((BLK, XH), lambda i: (i, 0)),
            pl.BlockSpec((BLK, XH), lambda i: (i, 0)),
            pl.BlockSpec((BLK, 1), lambda i: (i, 0)),
            pl.BlockSpec((BLK, 1), lambda i: (i, 0)),
            pl.BlockSpec((1, 1, BLK), lambda i: (i, 0, 0)),
            pl.BlockSpec((P, P), lambda i: (0, 0)),
            pl.BlockSpec((XD, P), lambda i: (0, 0)),
            pl.BlockSpec((1, P), lambda i: (0, 0)),
            pl.BlockSpec((P, P), lambda i: (0, 0)),
            pl.BlockSpec((1, P), lambda i: (0, 0)),
            pl.BlockSpec((P, 8), lambda i: (0, 0)),
            pl.BlockSpec((1, 8), lambda i: (0, 0)),
        ],
        out_specs=pl.BlockSpec((NG, 8), lambda i: (0, 0)),
        out_shape=jax.ShapeDtypeStruct((NG, 8), jnp.float32),
        scratch_shapes=[pltpu.VMEM((NG, P + 1), jnp.float32)],
    )(sa, sb, axa, axb, invd, nw, batch3, Wfh, Wfx, bf, Wm1, bm1, Wm2, bm2)


def _padw(W, r, c):
    return jnp.pad(W, ((0, r - W.shape[0]), (0, c - W.shape[1])))


def kernel(x, edge_index, edge_attr, edge_weight, node_weight, batch,
           W0, b0, W1, b1, W2, b2, W3, b3, Wf, bf, Wm1, bm1, Wm2, bm2):
    src = edge_index[0]
    dst = edge_index[1]

    W0x = _padw(W0[:XD], XD, P)
    W0e = _padw(W0[XD:], 16, P)
    b0p = jnp.pad(b0, (0, P - b0.shape[0])).reshape(1, P)
    W1p = _padw(W1, P, P)
    W2p = _padw(W2, P, P)
    W3p = _padw(W3, P, P)
    b1p = jnp.pad(b1, (0, P - b1.shape[0])).reshape(1, P)
    b2p = jnp.pad(b2, (0, P - b2.shape[0])).reshape(1, P)
    b3p = jnp.pad(b3, (0, P - b3.shape[0])).reshape(1, P)
    Wfh = _padw(Wf[:300], P, P)
    Wfx = _padw(Wf[300:], XD, P)
    bfp = jnp.pad(bf, (0, P - bf.shape[0])).reshape(1, P)
    Wm1p = _padw(Wm1, P, P)
    bm1p = jnp.pad(bm1, (0, P - bm1.shape[0])).reshape(1, P)
    Wm2p = _padw(Wm2, P, 8)
    bm2p = jnp.pad(bm2.reshape(1, 1), ((0, 0), (0, 7)))

    z32 = jnp.zeros((N, 32), jnp.float32)
    z64 = jnp.zeros((N, XH), jnp.float32)
    z160 = jnp.zeros((N, HD), jnp.float32)

    xa = x[:, :XH]
    xb = x[:, XH:]
    nw = node_weight.reshape(N, 1)
    batch3 = batch.reshape(N // BLK, 1, BLK)

    incp = _sc_inc_cnt(edge_attr, edge_weight, dst, z32)
    h0a, h0b, invd = _tc1(x, incp, W0x, W0e, b0p)

    axa, axb = _sc_gather_scatter(xa, xb, src, dst, edge_weight, z64, XH, 320)

    ha, hb = h0a, h0b
    for Wp, bp in ((W1p, b1p), (W2p, b2p), (W3p, b3p)):
        sa, sb = _sc_gather_scatter(ha, hb, src, dst, edge_weight, z160,
                                    HD, 160)
        ha, hb = _tc2(sa, sb, invd, h0a, h0b, Wp, bp)

    s4a, s4b = _sc_gather_scatter(ha, hb, src, dst, edge_weight, z160,
                                  HD, 160)
    out = _tc3(s4a, s4b, axa, axb, invd, nw, batch3,
               Wfh, Wfx, bfp, Wm1p, bm1p, Wm2p, bm2p)
    return out[:, 0]


# speedup vs baseline: 5.1564x; 1.5003x over previous
import dataclasses
import functools

import jax
import jax.numpy as jnp
from jax import lax
from jax.experimental import pallas as pl
from jax.experimental.pallas import tpu as pltpu
from jax.experimental.pallas import tpu_sc as plsc

N = 10000
E = 320000
P = 320
HD = P // 2
XD = 128
XH = XD // 2
NG = 32
NS = 16
BLK = 1000
HI = jax.lax.Precision.HIGHEST

_mesh = plsc.VectorSubcoreMesh(core_axis_name="c", subcore_axis_name="s")


def _sc_compiler_params():
    cp = pltpu.CompilerParams(use_tc_tiling_on_sc=False)
    if "needs_layout_passes" in pltpu.CompilerParams.__dataclass_fields__:
        cp = dataclasses.replace(cp, needs_layout_passes=False)
    return cp


def _sc_inc_cnt(edge_attr, edge_weight, dst, zeros32):
    K = 400
    EPW = E // (2 * NS)
    NCH = EPW // K

    def body(ea_hbm, ew_hbm, dst_hbm, z_hbm, out_hbm,
             eabuf, ewbuf, dstbuf, rows, acc, sem):
        c = lax.axis_index("c")
        s = lax.axis_index("s")

        @pl.when(s < 10)
        def _():
            pltpu.sync_copy(z_hbm.at[pl.ds(s * 1000, 1000)],
                            acc.at[pl.ds(s * 1000, 1000)])
        plsc.subcore_barrier()

        onevec = jnp.where(lax.iota(jnp.int32, 16) == 0,
                           jnp.float32(1.0), jnp.float32(0.0))
        base = (c * NS + s) * EPW

        @pl.loop(0, NCH)
        def _(i):
            off = base + i * K
            pltpu.sync_copy(ea_hbm.at[pl.ds(off, K)], eabuf)
            pltpu.sync_copy(ew_hbm.at[pl.ds(off, K)], ewbuf)
            pltpu.sync_copy(dst_hbm.at[pl.ds(off, K)], dstbuf)

            @pl.loop(0, K // 16)
            def _(g):
                wv = ewbuf[pl.ds(g * 16, 16)]
                for j in range(16):
                    e = g * 16 + j
                    rows[e, 0:16] = eabuf[e, :] * wv[j]
                    rows[e, 16:32] = onevec

            pltpu.sync_copy(rows, acc.at[dstbuf], add=True)

        plsc.subcore_barrier()

        @pl.when(s < 10)
        def _():
            pltpu.sync_copy(acc.at[pl.ds(s * 1000, 1000)],
                            out_hbm.at[c, pl.ds(s * 1000, 1000)])

    f = pl.kernel(
        body,
        out_type=jax.ShapeDtypeStruct((2, N, 32), jnp.float32),
        mesh=_mesh,
        scratch_types=[
            pltpu.VMEM((K, 16), jnp.float32),
            pltpu.VMEM((K,), jnp.float32),
            pltpu.VMEM((K,), jnp.int32),
            pltpu.VMEM((K, 32), jnp.float32),
            pltpu.VMEM_SHARED((N, 32), jnp.float32),
            pltpu.SemaphoreType.DMA,
        ],
        compiler_params=_sc_compiler_params(),
    )
    return f(edge_attr, edge_weight, dst, zeros32)


def _sc_gather_scatter(ha, hb, src, dst, edge_weight, zeros, D, K):
    EPW = E // NS
    NCH = EPW // K
    assert NCH * K == EPW and K % 16 == 0 and NCH >= 4

    def body(ha_hbm, hb_hbm, src_hbm, dst_hbm, ew_hbm, z_hbm,
             oa_hbm, ob_hbm,
             srcb0, srcb1, dstb0, dstb1, ewb0, ewb1, rows0, rows1,
             acc, gsem0, gsem1, isem):
        c = lax.axis_index("c")
        s = lax.axis_index("s")

        @pl.when(s < 10)
        def _():
            pltpu.sync_copy(z_hbm.at[pl.ds(s * 1000, 1000)],
                            acc.at[pl.ds(s * 1000, 1000)])
        plsc.subcore_barrier()

        srcbufs = (srcb0, srcb1)
        dstbufs = (dstb0, dstb1)
        ewbufs = (ewb0, ewb1)
        rowsbufs = (rows0, rows1)
        gsems = (gsem0, gsem1)
        base = s * EPW

        def run(h_hbm):
            def fetch_idx(i, b, is_sync):
                off = base + i * K
                cp = pltpu.sync_copy if is_sync else (
                    lambda a, bb: pltpu.async_copy(a, bb, isem))
                cp(src_hbm.at[pl.ds(off, K)], srcbufs[b])
                cp(dst_hbm.at[pl.ds(off, K)], dstbufs[b])
                cp(ew_hbm.at[pl.ds(off, K)], ewbufs[b])

            def wait_idx(i, b):
                off = base + i * K
                pltpu.make_async_copy(
                    src_hbm.at[pl.ds(off, K)], srcbufs[b], isem).wait()
                pltpu.make_async_copy(
                    dst_hbm.at[pl.ds(off, K)], dstbufs[b], isem).wait()
                pltpu.make_async_copy(
                    ew_hbm.at[pl.ds(off, K)], ewbufs[b], isem).wait()

            def gather_start(b):
                pltpu.async_copy(h_hbm.at[srcbufs[b]], rowsbufs[b], gsems[b])

            def gather_wait(b):
                pltpu.make_async_copy(
                    h_hbm.at[srcbufs[b]], rowsbufs[b], gsems[b]).wait()

            def scale_scatter(b):
                rows = rowsbufs[b]
                ewb = ewbufs[b]

                @pl.loop(0, K // 16)
                def _(g):
                    wv = ewb[pl.ds(g * 16, 16)]
                    for j in range(16):
                        e = g * 16 + j
                        w = wv[j]
                        for d in range(D // 16):
                            rows[e, d * 16:(d + 1) * 16] = (
                                rows[e, d * 16:(d + 1) * 16] * w)

                pltpu.sync_copy(rows, acc.at[dstbufs[b]], add=True)

            fetch_idx(0, 0, True)
            gather_start(0)
            fetch_idx(1, 1, False)

            def step(i, b, guard_prefetch):
                nb = 1 - b
                wait_idx(i + 1, nb)
                gather_start(nb)
                gather_wait(b)
                scale_scatter(b)
                if guard_prefetch:
                    @pl.when(i + 2 < NCH)
                    def _():
                        fetch_idx(i + 2, b, False)
                else:
                    fetch_idx(i + 2, b, False)

            @pl.loop(0, 2 * ((NCH - 1) // 2), step=2)
            def _(i):
                step(i, 0, False)
                step(i + 1, 1, True)

            if NCH % 2 == 1:
                gather_wait(0)
                scale_scatter(0)
            else:
                wait_idx(NCH - 1, 1)
                gather_start(1)
                gather_wait(0)
                scale_scatter(0)
                gather_wait(1)
                scale_scatter(1)

        @pl.when(c == 0)
        def _():
            run(ha_hbm)

        @pl.when(c == 1)
        def _():
            run(hb_hbm)

        plsc.subcore_barrier()

        @pl.when((s < 10) & (c == 0))
        def _():
            pltpu.sync_copy(acc.at[pl.ds(s * 1000, 1000)],
                            oa_hbm.at[pl.ds(s * 1000, 1000)])

        @pl.when((s < 10) & (c == 1))
        def _():
            pltpu.sync_copy(acc.at[pl.ds(s * 1000, 1000)],
                            ob_hbm.at[pl.ds(s * 1000, 1000)])

    f = pl.kernel(
        body,
        out_type=(jax.ShapeDtypeStruct((N, D), jnp.float32),
                  jax.ShapeDtypeStruct((N, D), jnp.float32)),
        mesh=_mesh,
        scratch_types=[
            pltpu.VMEM((K,), jnp.int32),
            pltpu.VMEM((K,), jnp.int32),
            pltpu.VMEM((K,), jnp.int32),
            pltpu.VMEM((K,), jnp.int32),
            pltpu.VMEM((K,), jnp.float32),
            pltpu.VMEM((K,), jnp.float32),
            pltpu.VMEM((K, D), jnp.float32),
            pltpu.VMEM((K, D), jnp.float32),
            pltpu.VMEM_SHARED((N, D), jnp.float32),
            pltpu.SemaphoreType.DMA,
            pltpu.SemaphoreType.DMA,
            pltpu.SemaphoreType.DMA,
        ],
        compiler_params=_sc_compiler_params(),
    )
    return f(ha, hb, src, dst, edge_weight, zeros)


def _dot(a, b):
    return lax.dot_general(a, b, (((1,), (0,)), ((), ())),
                           preferred_element_type=jnp.float32, precision=None)


def _tc1_kernel(x_ref, incp_ref, w0x_ref, w0e_ref, b0_ref,
                h0a_ref, h0b_ref, invd_ref):
    inc = incp_ref[0] + incp_ref[1]
    h = jnp.maximum(_dot(x_ref[...], w0x_ref[...])
                    + _dot(inc[:, :16], w0e_ref[...]) + b0_ref[...], 0.0)
    h0a_ref[...] = h[:, :HD]
    h0b_ref[...] = h[:, HD:]
    invd_ref[...] = 1.0 / jnp.maximum(inc[:, 16:17], 1.0)


def _tc1(x, incp, W0x, W0e, b0):
    grid = N // BLK
    return pl.pallas_call(
        _tc1_kernel,
        grid=(grid,),
        in_specs=[
            pl.BlockSpec((BLK, XD), lambda i: (i, 0)),
            pl.BlockSpec((2, BLK, 32), lambda i: (0, i, 0)),
            pl.BlockSpec((XD, P), lambda i: (0, 0)),
            pl.BlockSpec((16, P), lambda i: (0, 0)),
            pl.BlockSpec((1, P), lambda i: (0, 0)),
        ],
        out_specs=(
            pl.BlockSpec((BLK, HD), lambda i: (i, 0)),
            pl.BlockSpec((BLK, HD), lambda i: (i, 0)),
            pl.BlockSpec((BLK, 1), lambda i: (i, 0)),
        ),
        out_shape=(
            jax.ShapeDtypeStruct((N, HD), jnp.float32),
            jax.ShapeDtypeStruct((N, HD), jnp.float32),
            jax.ShapeDtypeStruct((N, 1), jnp.float32),
        ),
    )(x, incp, W0x, W0e, b0)


def _tc2_kernel(sa_ref, sb_ref, invd_ref, h0a_ref, h0b_ref, w_ref, b_ref,
                ha_ref, hb_ref):
    aggr = jnp.concatenate([sa_ref[...], sb_ref[...]], axis=1) * invd_ref[...]
    h0 = jnp.concatenate([h0a_ref[...], h0b_ref[...]], axis=1)
    h = jnp.maximum(h0 + _dot(aggr, w_ref[...]) + b_ref[...], 0.0)
    ha_ref[...] = h[:, :HD]
    hb_ref[...] = h[:, HD:]


def _tc2(sa, sb, invd, h0a, h0b, W, b):
    grid = N // BLK
    return pl.pallas_call(
        _tc2_kernel,
        grid=(grid,),
        in_specs=[
            pl.BlockSpec((BLK, HD), lambda i: (i, 0)),
            pl.BlockSpec((BLK, HD), lambda i: (i, 0)),
            pl.BlockSpec((BLK, 1), lambda i: (i, 0)),
            pl.BlockSpec((BLK, HD), lambda i: (i, 0)),
            pl.BlockSpec((BLK, HD), lambda i: (i, 0)),
            pl.BlockSpec((P, P), lambda i: (0, 0)),
            pl.BlockSpec((1, P), lambda i: (0, 0)),
        ],
        out_specs=(
            pl.BlockSpec((BLK, HD), lambda i: (i, 0)),
            pl.BlockSpec((BLK, HD), lambda i: (i, 0)),
        ),
        out_shape=(
            jax.ShapeDtypeStruct((N, HD), jnp.float32),
            jax.ShapeDtypeStruct((N, HD), jnp.float32),
        ),
    )(sa, sb, invd, h0a, h0b, W, b)


def _tc3_kernel(sa_ref, sb_ref, axa_ref, axb_ref, invd_ref, nw_ref, bt_ref,
                wfh_ref, wfx_ref, bf_ref, wm1_ref, bm1_ref, wm2_ref, bm2_ref,
                out_ref, acc):
    i = pl.program_id(0)
    n = pl.num_programs(0)

    @pl.when(i == 0)
    def _():
        acc[...] = jnp.zeros_like(acc)

    invd = invd_ref[...]
    aggh = jnp.concatenate([sa_ref[...], sb_ref[...]], axis=1) * invd
    aggx = jnp.concatenate([axa_ref[...], axb_ref[...]], axis=1) * invd
    h = jnp.maximum(_dot(aggh, wfh_ref[...]) + _dot(aggx, wfx_ref[...])
                    + bf_ref[...], 0.0)
    hw = h * nw_ref[...]
    bids = bt_ref[0, 0, :]
    onehot = (jax.lax.broadcasted_iota(jnp.int32, (NG, BLK), 0)
              == bids[None, :]).astype(jnp.float32)
    acc[:, :P] += _dot(onehot, hw)
    acc[:, P:] += jnp.sum(onehot, axis=1, keepdims=True)

    @pl.when(i == n - 1)
    def _():
        ge = acc[:, :P] / jnp.maximum(acc[:, P:], 1.0)
        hm = jnp.maximum(_dot(ge, wm1_ref[...]) + bm1_ref[...], 0.0)
        out_ref[...] = _dot(hm, wm2_ref[...]) + bm2_ref[...]


def _tc3(sa, sb, axa, axb, invd, nw, batch3, Wfh, Wfx, bf, Wm1, bm1, Wm2, bm2):
    grid = N // BLK
    return pl.pallas_call(
        _tc3_kernel,
        grid=(grid,),
        in_specs=[
            pl.BlockSpec((BLK, HD), lambda i: (i, 0)),
            pl.BlockSpec((BLK, HD), lambda i: (i, 0)),
            pl.BlockSpec((BLK, XH), lambda i: (i, 0)),
            pl.BlockSpec((BLK, XH), lambda i: (i, 0)),
            pl.BlockSpec((BLK, 1), lambda i: (i, 0)),
            pl.BlockSpec((BLK, 1), lambda i: (i, 0)),
            pl.BlockSpec((1, 1, BLK), lambda i: (i, 0, 0)),
            pl.BlockSpec((P, P), lambda i: (0, 0)),
            pl.BlockSpec((XD, P), lambda i: (0, 0)),
            pl.BlockSpec((1, P), lambda i: (0, 0)),
            pl.BlockSpec((P, P), lambda i: (0, 0)),
            pl.BlockSpec((1, P), lambda i: (0, 0)),
            pl.BlockSpec((P, 8), lambda i: (0, 0)),
            pl.BlockSpec((1, 8), lambda i: (0, 0)),
        ],
        out_specs=pl.BlockSpec((NG, 8), lambda i: (0, 0)),
        out_shape=jax.ShapeDtypeStruct((NG, 8), jnp.float32),
        scratch_shapes=[pltpu.VMEM((NG, P + 1), jnp.float32)],
    )(sa, sb, axa, axb, invd, nw, batch3, Wfh, Wfx, bf, Wm1, bm1, Wm2, bm2)


def _padw(W, r, c):
    return jnp.pad(W, ((0, r - W.shape[0]), (0, c - W.shape[1])))


def kernel(x, edge_index, edge_attr, edge_weight, node_weight, batch,
           W0, b0, W1, b1, W2, b2, W3, b3, Wf, bf, Wm1, bm1, Wm2, bm2):
    src = edge_index[0]
    dst = edge_index[1]

    W0x = _padw(W0[:XD], XD, P)
    W0e = _padw(W0[XD:], 16, P)
    b0p = jnp.pad(b0, (0, P - b0.shape[0])).reshape(1, P)
    W1p = _padw(W1, P, P)
    W2p = _padw(W2, P, P)
    W3p = _padw(W3, P, P)
    b1p = jnp.pad(b1, (0, P - b1.shape[0])).reshape(1, P)
    b2p = jnp.pad(b2, (0, P - b2.shape[0])).reshape(1, P)
    b3p = jnp.pad(b3, (0, P - b3.shape[0])).reshape(1, P)
    Wfh = _padw(Wf[:300], P, P)
    Wfx = _padw(Wf[300:], XD, P)
    bfp = jnp.pad(bf, (0, P - bf.shape[0])).reshape(1, P)
    Wm1p = _padw(Wm1, P, P)
    bm1p = jnp.pad(bm1, (0, P - bm1.shape[0])).reshape(1, P)
    Wm2p = _padw(Wm2, P, 8)
    bm2p = jnp.pad(bm2.reshape(1, 1), ((0, 0), (0, 7)))

    z32 = jnp.zeros((N, 32), jnp.float32)
    z64 = jnp.zeros((N, XH), jnp.float32)
    z160 = jnp.zeros((N, HD), jnp.float32)

    xa = x[:, :XH]
    xb = x[:, XH:]
    nw = node_weight.reshape(N, 1)
    batch3 = batch.reshape(N // BLK, 1, BLK)

    DEBUG_XLA_SEG = False

    def _xla_gs(ha_, hb_, D):
        h = jnp.concatenate([ha_, hb_], axis=1)
        s = jax.ops.segment_sum(edge_weight[:, None] * jnp.take(h, src, axis=0),
                                dst, num_segments=N)
        return s[:, :D], s[:, D:]

    incp = _sc_inc_cnt(edge_attr, edge_weight, dst, z32)
    if DEBUG_XLA_SEG:
        inc_x = jax.ops.segment_sum(edge_weight[:, None] * edge_attr, dst,
                                    num_segments=N)
        cnt_x = jax.ops.segment_sum(jnp.ones_like(edge_weight), dst,
                                    num_segments=N)
        incp = jnp.zeros((2, N, 32), jnp.float32)
        incp = incp.at[0, :, :16].set(inc_x).at[0, :, 16].set(cnt_x)
    h0a, h0b, invd = _tc1(x, incp, W0x, W0e, b0p)

    if DEBUG_XLA_SEG:
        axa, axb = _xla_gs(xa, xb, XH)
    else:
        axa, axb = _sc_gather_scatter(xa, xb, src, dst, edge_weight, z64,
                                      XH, 160)

    ha, hb = h0a, h0b
    for Wp, bp in ((W1p, b1p), (W2p, b2p), (W3p, b3p)):
        if DEBUG_XLA_SEG:
            sa, sb = _xla_gs(ha, hb, HD)
        else:
            sa, sb = _sc_gather_scatter(ha, hb, src, dst, edge_weight, z160,
                                        HD, 80)
        ha, hb = _tc2(sa, sb, invd, h0a, h0b, Wp, bp)

    if DEBUG_XLA_SEG:
        s4a, s4b = _xla_gs(ha, hb, HD)
    else:
        s4a, s4b = _sc_gather_scatter(ha, hb, src, dst, edge_weight, z160,
                                      HD, 80)
    out = _tc3(s4a, s4b, axa, axb, invd, nw, batch3,
               Wfh, Wfx, bfp, Wm1p, bm1p, Wm2p, bm2p)
    return out[:, 0]


# speedup vs baseline: 5.2526x; 1.0187x over previous
import dataclasses
import functools

import jax
import jax.numpy as jnp
from jax import lax
from jax.experimental import pallas as pl
from jax.experimental.pallas import tpu as pltpu
from jax.experimental.pallas import tpu_sc as plsc

N = 10000
E = 320000
P = 320
HD = P // 2
XD = 128
XH = XD // 2
NG = 32
NS = 16
BLK = 1000
HI = jax.lax.Precision.HIGHEST

_mesh = plsc.VectorSubcoreMesh(core_axis_name="c", subcore_axis_name="s")


def _sc_compiler_params():
    cp = pltpu.CompilerParams(use_tc_tiling_on_sc=False)
    if "needs_layout_passes" in pltpu.CompilerParams.__dataclass_fields__:
        cp = dataclasses.replace(cp, needs_layout_passes=False)
    return cp


def _sc_inc_cnt(edge_attr, edge_weight, dst, zeros32):
    K = 400
    EPW = E // (2 * NS)
    NCH = EPW // K

    def body(ea_hbm, ew_hbm, dst_hbm, z_hbm, out_hbm,
             eabuf, ewbuf, dstbuf, rows, acc, sem):
        c = lax.axis_index("c")
        s = lax.axis_index("s")

        @pl.when(s < 10)
        def _():
            pltpu.sync_copy(z_hbm.at[pl.ds(s * 1000, 1000)],
                            acc.at[pl.ds(s * 1000, 1000)])
        plsc.subcore_barrier()

        onevec = jnp.where(lax.iota(jnp.int32, 16) == 0,
                           jnp.float32(1.0), jnp.float32(0.0))
        base = (c * NS + s) * EPW

        @pl.loop(0, NCH)
        def _(i):
            off = base + i * K
            pltpu.sync_copy(ea_hbm.at[pl.ds(off, K)], eabuf)
            pltpu.sync_copy(ew_hbm.at[pl.ds(off, K)], ewbuf)
            pltpu.sync_copy(dst_hbm.at[pl.ds(off, K)], dstbuf)

            @pl.loop(0, K // 16)
            def _(g):
                wv = ewbuf[pl.ds(g * 16, 16)]
                for j in range(16):
                    e = g * 16 + j
                    rows[e, 0:16] = eabuf[e, :] * wv[j]
                    rows[e, 16:32] = onevec

            pltpu.sync_copy(rows, acc.at[dstbuf], add=True)

        plsc.subcore_barrier()

        @pl.when(s < 10)
        def _():
            pltpu.sync_copy(acc.at[pl.ds(s * 1000, 1000)],
                            out_hbm.at[c, pl.ds(s * 1000, 1000)])

    f = pl.kernel(
        body,
        out_type=jax.ShapeDtypeStruct((2, N, 32), jnp.float32),
        mesh=_mesh,
        scratch_types=[
            pltpu.VMEM((K, 16), jnp.float32),
            pltpu.VMEM((K,), jnp.float32),
            pltpu.VMEM((K,), jnp.int32),
            pltpu.VMEM((K, 32), jnp.float32),
            pltpu.VMEM_SHARED((N, 32), jnp.float32),
            pltpu.SemaphoreType.DMA,
        ],
        compiler_params=_sc_compiler_params(),
    )
    return f(edge_attr, edge_weight, dst, zeros32)


def _sc_gather_scatter(ha, hb, src, dst, edge_weight, zeros, D, K):
    EPW = E // NS
    NCH = EPW // K
    assert NCH * K == EPW and K % 16 == 0 and NCH >= 4

    def body(ha_hbm, hb_hbm, src_hbm, dst_hbm, ew_hbm, z_hbm,
             oa_hbm, ob_hbm,
             srcb0, srcb1, dstb0, dstb1, ewb0, ewb1, rows0, rows1,
             acc, gsem0, gsem1, isem):
        c = lax.axis_index("c")
        s = lax.axis_index("s")

        @pl.when(s < 10)
        def _():
            pltpu.sync_copy(z_hbm.at[pl.ds(s * 1000, 1000)],
                            acc.at[pl.ds(s * 1000, 1000)])
        plsc.subcore_barrier()

        srcbufs = (srcb0, srcb1)
        dstbufs = (dstb0, dstb1)
        ewbufs = (ewb0, ewb1)
        rowsbufs = (rows0, rows1)
        gsems = (gsem0, gsem1)
        base = s * EPW

        def run(h_hbm):
            def fetch_idx(i, b, is_sync):
                off = base + i * K
                cp = pltpu.sync_copy if is_sync else (
                    lambda a, bb: pltpu.async_copy(a, bb, isem))
                cp(src_hbm.at[pl.ds(off, K)], srcbufs[b])
                cp(dst_hbm.at[pl.ds(off, K)], dstbufs[b])
                cp(ew_hbm.at[pl.ds(off, K)], ewbufs[b])

            def wait_idx(i, b):
                off = base + i * K
                pltpu.make_async_copy(
                    src_hbm.at[pl.ds(off, K)], srcbufs[b], isem).wait()
                pltpu.make_async_copy(
                    dst_hbm.at[pl.ds(off, K)], dstbufs[b], isem).wait()
                pltpu.make_async_copy(
                    ew_hbm.at[pl.ds(off, K)], ewbufs[b], isem).wait()

            def gather_start(b):
                pltpu.async_copy(h_hbm.at[srcbufs[b]], rowsbufs[b], gsems[b])

            def gather_wait(b):
                pltpu.make_async_copy(
                    h_hbm.at[srcbufs[b]], rowsbufs[b], gsems[b]).wait()

            def scale_scatter(b):
                rows = rowsbufs[b]
                ewb = ewbufs[b]

                @plsc.parallel_loop(0, K // 16, 1, unroll=2)
                def _(g):
                    wv = ewb[pl.ds(g * 16, 16)]
                    for j in range(16):
                        e = g * 16 + j
                        w = wv[j]
                        for d in range(D // 16):
                            rows[e, d * 16:(d + 1) * 16] = (
                                rows[e, d * 16:(d + 1) * 16] * w)

                pltpu.sync_copy(rows, acc.at[dstbufs[b]], add=True)

            fetch_idx(0, 0, True)
            gather_start(0)
            fetch_idx(1, 1, False)

            def step(i, b, guard_prefetch):
                nb = 1 - b
                wait_idx(i + 1, nb)
                gather_start(nb)
                gather_wait(b)
                scale_scatter(b)
                if guard_prefetch:
                    @pl.when(i + 2 < NCH)
                    def _():
                        fetch_idx(i + 2, b, False)
                else:
                    fetch_idx(i + 2, b, False)

            @pl.loop(0, 2 * ((NCH - 1) // 2), step=2)
            def _(i):
                step(i, 0, False)
                step(i + 1, 1, True)

            if NCH % 2 == 1:
                gather_wait(0)
                scale_scatter(0)
            else:
                wait_idx(NCH - 1, 1)
                gather_start(1)
                gather_wait(0)
                scale_scatter(0)
                gather_wait(1)
                scale_scatter(1)

        @pl.when(c == 0)
        def _():
            run(ha_hbm)

        @pl.when(c == 1)
        def _():
            run(hb_hbm)

        plsc.subcore_barrier()

        @pl.when((s < 10) & (c == 0))
        def _():
            pltpu.sync_copy(acc.at[pl.ds(s * 1000, 1000)],
                            oa_hbm.at[pl.ds(s * 1000, 1000)])

        @pl.when((s < 10) & (c == 1))
        def _():
            pltpu.sync_copy(acc.at[pl.ds(s * 1000, 1000)],
                            ob_hbm.at[pl.ds(s * 1000, 1000)])

    f = pl.kernel(
        body,
        out_type=(jax.ShapeDtypeStruct((N, D), jnp.float32),
                  jax.ShapeDtypeStruct((N, D), jnp.float32)),
        mesh=_mesh,
        scratch_types=[
            pltpu.VMEM((K,), jnp.int32),
            pltpu.VMEM((K,), jnp.int32),
            pltpu.VMEM((K,), jnp.int32),
            pltpu.VMEM((K,), jnp.int32),
            pltpu.VMEM((K,), jnp.float32),
            pltpu.VMEM((K,), jnp.float32),
            pltpu.VMEM((K, D), jnp.float32),
            pltpu.VMEM((K, D), jnp.float32),
            pltpu.VMEM_SHARED((N, D), jnp.float32),
            pltpu.SemaphoreType.DMA,
            pltpu.SemaphoreType.DMA,
            pltpu.SemaphoreType.DMA,
        ],
        compiler_params=_sc_compiler_params(),
    )
    return f(ha, hb, src, dst, edge_weight, zeros)


def _dot(a, b):
    return lax.dot_general(a, b, (((1,), (0,)), ((), ())),
                           preferred_element_type=jnp.float32, precision=None)


def _tc1_kernel(x_ref, incp_ref, w0x_ref, w0e_ref, b0_ref,
                h0a_ref, h0b_ref, invd_ref):
    inc = incp_ref[0] + incp_ref[1]
    h = jnp.maximum(_dot(x_ref[...], w0x_ref[...])
                    + _dot(inc[:, :16], w0e_ref[...]) + b0_ref[...], 0.0)
    h0a_ref[...] = h[:, :HD]
    h0b_ref[...] = h[:, HD:]
    invd_ref[...] = 1.0 / jnp.maximum(inc[:, 16:17], 1.0)


def _tc1(x, incp, W0x, W0e, b0):
    grid = N // BLK
    return pl.pallas_call(
        _tc1_kernel,
        grid=(grid,),
        in_specs=[
            pl.BlockSpec((BLK, XD), lambda i: (i, 0)),
            pl.BlockSpec((2, BLK, 32), lambda i: (0, i, 0)),
            pl.BlockSpec((XD, P), lambda i: (0, 0)),
            pl.BlockSpec((16, P), lambda i: (0, 0)),
            pl.BlockSpec((1, P), lambda i: (0, 0)),
        ],
        out_specs=(
            pl.BlockSpec((BLK, HD), lambda i: (i, 0)),
            pl.BlockSpec((BLK, HD), lambda i: (i, 0)),
            pl.BlockSpec((BLK, 1), lambda i: (i, 0)),
        ),
        out_shape=(
            jax.ShapeDtypeStruct((N, HD), jnp.float32),
            jax.ShapeDtypeStruct((N, HD), jnp.float32),
            jax.ShapeDtypeStruct((N, 1), jnp.float32),
        ),
    )(x, incp, W0x, W0e, b0)


def _tc2_kernel(sa_ref, sb_ref, invd_ref, h0a_ref, h0b_ref, w_ref, b_ref,
                ha_ref, hb_ref):
    aggr = jnp.concatenate([sa_ref[...], sb_ref[...]], axis=1) * invd_ref[...]
    h0 = jnp.concatenate([h0a_ref[...], h0b_ref[...]], axis=1)
    h = jnp.maximum(h0 + _dot(aggr, w_ref[...]) + b_ref[...], 0.0)
    ha_ref[...] = h[:, :HD]
    hb_ref[...] = h[:, HD:]


def _tc2(sa, sb, invd, h0a, h0b, W, b):
    grid = N // BLK
    return pl.pallas_call(
        _tc2_kernel,
        grid=(grid,),
        in_specs=[
            pl.BlockSpec((BLK, HD), lambda i: (i, 0)),
            pl.BlockSpec((BLK, HD), lambda i: (i, 0)),
            pl.BlockSpec((BLK, 1), lambda i: (i, 0)),
            pl.BlockSpec((BLK, HD), lambda i: (i, 0)),
            pl.BlockSpec((BLK, HD), lambda i: (i, 0)),
            pl.BlockSpec((P, P), lambda i: (0, 0)),
            pl.BlockSpec((1, P), lambda i: (0, 0)),
        ],
        out_specs=(
            pl.BlockSpec((BLK, HD), lambda i: (i, 0)),
            pl.BlockSpec((BLK, HD), lambda i: (i, 0)),
        ),
        out_shape=(
            jax.ShapeDtypeStruct((N, HD), jnp.float32),
            jax.ShapeDtypeStruct((N, HD), jnp.float32),
        ),
    )(sa, sb, invd, h0a, h0b, W, b)


def _tc3_kernel(sa_ref, sb_ref, axa_ref, axb_ref, invd_ref, nw_ref, bt_ref,
                wfh_ref, wfx_ref, bf_ref, wm1_ref, bm1_ref, wm2_ref, bm2_ref,
                out_ref, acc):
    i = pl.program_id(0)
    n = pl.num_programs(0)

    @pl.when(i == 0)
    def _():
        acc[...] = jnp.zeros_like(acc)

    invd = invd_ref[...]
    aggh = jnp.concatenate([sa_ref[...], sb_ref[...]], axis=1) * invd
    aggx = jnp.concatenate([axa_ref[...], axb_ref[...]], axis=1) * invd
    h = jnp.maximum(_dot(aggh, wfh_ref[...]) + _dot(aggx, wfx_ref[...])
                    + bf_ref[...], 0.0)
    hw = h * nw_ref[...]
    bids = bt_ref[0, 0, :]
    onehot = (jax.lax.broadcasted_iota(jnp.int32, (NG, BLK), 0)
              == bids[None, :]).astype(jnp.float32)
    acc[:, :P] += _dot(onehot, hw)
    acc[:, P:] += jnp.sum(onehot, axis=1, keepdims=True)

    @pl.when(i == n - 1)
    def _():
        ge = acc[:, :P] / jnp.maximum(acc[:, P:], 1.0)
        hm = jnp.maximum(_dot(ge, wm1_ref[...]) + bm1_ref[...], 0.0)
        out_ref[...] = _dot(hm, wm2_ref[...]) + bm2_ref[...]


def _tc3(sa, sb, axa, axb, invd, nw, batch3, Wfh, Wfx, bf, Wm1, bm1, Wm2, bm2):
    grid = N // BLK
    return pl.pallas_call(
        _tc3_kernel,
        grid=(grid,),
        in_specs=[
            pl.BlockSpec((BLK, HD), lambda i: (i, 0)),
            pl.BlockSpec((BLK, HD), lambda i: (i, 0)),
            pl.BlockSpec((BLK, XH), lambda i: (i, 0)),
            pl.BlockSpec((BLK, XH), lambda i: (i, 0)),
            pl.BlockSpec((BLK, 1), lambda i: (i, 0)),
            pl.BlockSpec((BLK, 1), lambda i: (i, 0)),
            pl.BlockSpec((1, 1, BLK), lambda i: (i, 0, 0)),
            pl.BlockSpec((P, P), lambda i: (0, 0)),
            pl.BlockSpec((XD, P), lambda i: (0, 0)),
            pl.BlockSpec((1, P), lambda i: (0, 0)),
            pl.BlockSpec((P, P), lambda i: (0, 0)),
            pl.BlockSpec((1, P), lambda i: (0, 0)),
            pl.BlockSpec((P, 8), lambda i: (0, 0)),
            pl.BlockSpec((1, 8), lambda i: (0, 0)),
        ],
        out_specs=pl.BlockSpec((NG, 8), lambda i: (0, 0)),
        out_shape=jax.ShapeDtypeStruct((NG, 8), jnp.float32),
        scratch_shapes=[pltpu.VMEM((NG, P + 1), jnp.float32)],
    )(sa, sb, axa, axb, invd, nw, batch3, Wfh, Wfx, bf, Wm1, bm1, Wm2, bm2)


def _padw(W, r, c):
    return jnp.pad(W, ((0, r - W.shape[0]), (0, c - W.shape[1])))


def kernel(x, edge_index, edge_attr, edge_weight, node_weight, batch,
           W0, b0, W1, b1, W2, b2, W3, b3, Wf, bf, Wm1, bm1, Wm2, bm2):
    src = edge_index[0]
    dst = edge_index[1]

    W0x = _padw(W0[:XD], XD, P)
    W0e = _padw(W0[XD:], 16, P)
    b0p = jnp.pad(b0, (0, P - b0.shape[0])).reshape(1, P)
    W1p = _padw(W1, P, P)
    W2p = _padw(W2, P, P)
    W3p = _padw(W3, P, P)
    b1p = jnp.pad(b1, (0, P - b1.shape[0])).reshape(1, P)
    b2p = jnp.pad(b2, (0, P - b2.shape[0])).reshape(1, P)
    b3p = jnp.pad(b3, (0, P - b3.shape[0])).reshape(1, P)
    Wfh = _padw(Wf[:300], P, P)
    Wfx = _padw(Wf[300:], XD, P)
    bfp = jnp.pad(bf, (0, P - bf.shape[0])).reshape(1, P)
    Wm1p = _padw(Wm1, P, P)
    bm1p = jnp.pad(bm1, (0, P - bm1.shape[0])).reshape(1, P)
    Wm2p = _padw(Wm2, P, 8)
    bm2p = jnp.pad(bm2.reshape(1, 1), ((0, 0), (0, 7)))

    z32 = jnp.zeros((N, 32), jnp.float32)
    z64 = jnp.zeros((N, XH), jnp.float32)
    z160 = jnp.zeros((N, HD), jnp.float32)

    xa = x[:, :XH]
    xb = x[:, XH:]
    nw = node_weight.reshape(N, 1)
    batch3 = batch.reshape(N // BLK, 1, BLK)

    DEBUG_XLA_SEG = False

    def _xla_gs(ha_, hb_, D):
        h = jnp.concatenate([ha_, hb_], axis=1)
        s = jax.ops.segment_sum(edge_weight[:, None] * jnp.take(h, src, axis=0),
                                dst, num_segments=N)
        return s[:, :D], s[:, D:]

    incp = _sc_inc_cnt(edge_attr, edge_weight, dst, z32)
    if DEBUG_XLA_SEG:
        inc_x = jax.ops.segment_sum(edge_weight[:, None] * edge_attr, dst,
                                    num_segments=N)
        cnt_x = jax.ops.segment_sum(jnp.ones_like(edge_weight), dst,
                                    num_segments=N)
        incp = jnp.zeros((2, N, 32), jnp.float32)
        incp = incp.at[0, :, :16].set(inc_x).at[0, :, 16].set(cnt_x)
    h0a, h0b, invd = _tc1(x, incp, W0x, W0e, b0p)

    if DEBUG_XLA_SEG:
        axa, axb = _xla_gs(xa, xb, XH)
    else:
        axa, axb = _sc_gather_scatter(xa, xb, src, dst, edge_weight, z64,
                                      XH, 160)

    ha, hb = h0a, h0b
    for Wp, bp in ((W1p, b1p), (W2p, b2p), (W3p, b3p)):
        if DEBUG_XLA_SEG:
            sa, sb = _xla_gs(ha, hb, HD)
        else:
            sa, sb = _sc_gather_scatter(ha, hb, src, dst, edge_weight, z160,
                                        HD, 80)
        ha, hb = _tc2(sa, sb, invd, h0a, h0b, Wp, bp)

    if DEBUG_XLA_SEG:
        s4a, s4b = _xla_gs(ha, hb, HD)
    else:
        s4a, s4b = _sc_gather_scatter(ha, hb, src, dst, edge_weight, z160,
                                      HD, 80)
    out = _tc3(s4a, s4b, axa, axb, invd, nw, batch3,
               Wfh, Wfx, bfp, Wm1p, bm1p, Wm2p, bm2p)
    return out[:, 0]


# speedup vs baseline: 5.2534x; 1.0001x over previous
import dataclasses
import functools

import jax
import jax.numpy as jnp
from jax import lax
from jax.experimental import pallas as pl
from jax.experimental.pallas import tpu as pltpu
from jax.experimental.pallas import tpu_sc as plsc

N = 10000
E = 320000
P = 320
HD = P // 2
XD = 128
XH = XD // 2
NG = 32
NS = 16
BLK = 1000
HI = jax.lax.Precision.HIGHEST

_mesh = plsc.VectorSubcoreMesh(core_axis_name="c", subcore_axis_name="s")


def _sc_compiler_params():
    cp = pltpu.CompilerParams(use_tc_tiling_on_sc=False)
    if "needs_layout_passes" in pltpu.CompilerParams.__dataclass_fields__:
        cp = dataclasses.replace(cp, needs_layout_passes=False)
    return cp


def _sc_inc_cnt(edge_attr, edge_weight, dst, zeros32):
    K = 400
    EPW = E // (2 * NS)
    NCH = EPW // K

    def body(ea_hbm, ew_hbm, dst_hbm, z_hbm, out_hbm,
             eabuf, ewbuf, dstbuf, rows, acc, sem):
        c = lax.axis_index("c")
        s = lax.axis_index("s")

        @pl.when(s < 10)
        def _():
            pltpu.sync_copy(z_hbm.at[pl.ds(s * 1000, 1000)],
                            acc.at[pl.ds(s * 1000, 1000)])
        plsc.subcore_barrier()

        onevec = jnp.where(lax.iota(jnp.int32, 16) == 0,
                           jnp.float32(1.0), jnp.float32(0.0))
        base = (c * NS + s) * EPW

        @pl.loop(0, NCH)
        def _(i):
            off = base + i * K
            pltpu.sync_copy(ea_hbm.at[pl.ds(off, K)], eabuf)
            pltpu.sync_copy(ew_hbm.at[pl.ds(off, K)], ewbuf)
            pltpu.sync_copy(dst_hbm.at[pl.ds(off, K)], dstbuf)

            @pl.loop(0, K // 16)
            def _(g):
                wv = ewbuf[pl.ds(g * 16, 16)]
                for j in range(16):
                    e = g * 16 + j
                    rows[e, 0:16] = eabuf[e, :] * wv[j]
                    rows[e, 16:32] = onevec

            pltpu.sync_copy(rows, acc.at[dstbuf], add=True)

        plsc.subcore_barrier()

        @pl.when(s < 10)
        def _():
            pltpu.sync_copy(acc.at[pl.ds(s * 1000, 1000)],
                            out_hbm.at[c, pl.ds(s * 1000, 1000)])

    f = pl.kernel(
        body,
        out_type=jax.ShapeDtypeStruct((2, N, 32), jnp.float32),
        mesh=_mesh,
        scratch_types=[
            pltpu.VMEM((K, 16), jnp.float32),
            pltpu.VMEM((K,), jnp.float32),
            pltpu.VMEM((K,), jnp.int32),
            pltpu.VMEM((K, 32), jnp.float32),
            pltpu.VMEM_SHARED((N, 32), jnp.float32),
            pltpu.SemaphoreType.DMA,
        ],
        compiler_params=_sc_compiler_params(),
    )
    return f(edge_attr, edge_weight, dst, zeros32)


def _sc_gather_scatter(ha, hb, src, dst, edge_weight, zeros, D, K):
    EPW = E // NS
    NCH = EPW // K
    assert NCH * K == EPW and K % 16 == 0 and NCH >= 4

    def body(ha_hbm, hb_hbm, src_hbm, dst_hbm, ew_hbm, z_hbm,
             oa_hbm, ob_hbm,
             srcb0, srcb1, dstb0, dstb1, ewb0, ewb1, rows0, rows1,
             acc, gsem0, gsem1, isem):
        c = lax.axis_index("c")
        s = lax.axis_index("s")

        @pl.when(s < 10)
        def _():
            pltpu.sync_copy(z_hbm.at[pl.ds(s * 1000, 1000)],
                            acc.at[pl.ds(s * 1000, 1000)])
        plsc.subcore_barrier()

        srcbufs = (srcb0, srcb1)
        dstbufs = (dstb0, dstb1)
        ewbufs = (ewb0, ewb1)
        rowsbufs = (rows0, rows1)
        gsems = (gsem0, gsem1)
        base = s * EPW

        def run(h_hbm):
            def fetch_idx(i, b, is_sync):
                off = base + i * K
                cp = pltpu.sync_copy if is_sync else (
                    lambda a, bb: pltpu.async_copy(a, bb, isem))
                cp(src_hbm.at[pl.ds(off, K)], srcbufs[b])
                cp(dst_hbm.at[pl.ds(off, K)], dstbufs[b])
                cp(ew_hbm.at[pl.ds(off, K)], ewbufs[b])

            def wait_idx(i, b):
                off = base + i * K
                pltpu.make_async_copy(
                    src_hbm.at[pl.ds(off, K)], srcbufs[b], isem).wait()
                pltpu.make_async_copy(
                    dst_hbm.at[pl.ds(off, K)], dstbufs[b], isem).wait()
                pltpu.make_async_copy(
                    ew_hbm.at[pl.ds(off, K)], ewbufs[b], isem).wait()

            def gather_start(b):
                pltpu.async_copy(h_hbm.at[srcbufs[b]], rowsbufs[b], gsems[b])

            def gather_wait(b):
                pltpu.make_async_copy(
                    h_hbm.at[srcbufs[b]], rowsbufs[b], gsems[b]).wait()

            def scale_scatter(b):
                rows = rowsbufs[b]
                ewb = ewbufs[b]

                @plsc.parallel_loop(0, K // 16, 1, unroll=2)
                def _(g):
                    wv = ewb[pl.ds(g * 16, 16)]
                    for j in range(16):
                        e = g * 16 + j
                        w = wv[j]
                        for d in range(D // 16):
                            rows[e, d * 16:(d + 1) * 16] = (
                                rows[e, d * 16:(d + 1) * 16] * w)

                pltpu.sync_copy(rows, acc.at[dstbufs[b]], add=True)

            fetch_idx(0, 0, True)
            gather_start(0)
            fetch_idx(1, 1, False)

            def step(i, b, guard_prefetch):
                nb = 1 - b
                wait_idx(i + 1, nb)
                gather_start(nb)
                gather_wait(b)
                scale_scatter(b)
                if guard_prefetch:
                    @pl.when(i + 2 < NCH)
                    def _():
                        fetch_idx(i + 2, b, False)
                else:
                    fetch_idx(i + 2, b, False)

            @pl.loop(0, 2 * ((NCH - 1) // 2), step=2)
            def _(i):
                step(i, 0, False)
                step(i + 1, 1, True)

            if NCH % 2 == 1:
                gather_wait(0)
                scale_scatter(0)
            else:
                wait_idx(NCH - 1, 1)
                gather_start(1)
                gather_wait(0)
                scale_scatter(0)
                gather_wait(1)
                scale_scatter(1)

        @pl.when(c == 0)
        def _():
            run(ha_hbm)

        @pl.when(c == 1)
        def _():
            run(hb_hbm)

        plsc.subcore_barrier()

        @pl.when((s < 10) & (c == 0))
        def _():
            pltpu.sync_copy(acc.at[pl.ds(s * 1000, 1000)],
                            oa_hbm.at[pl.ds(s * 1000, 1000)])

        @pl.when((s < 10) & (c == 1))
        def _():
            pltpu.sync_copy(acc.at[pl.ds(s * 1000, 1000)],
                            ob_hbm.at[pl.ds(s * 1000, 1000)])

    f = pl.kernel(
        body,
        out_type=(jax.ShapeDtypeStruct((N, D), jnp.float32),
                  jax.ShapeDtypeStruct((N, D), jnp.float32)),
        mesh=_mesh,
        scratch_types=[
            pltpu.VMEM((K,), jnp.int32),
            pltpu.VMEM((K,), jnp.int32),
            pltpu.VMEM((K,), jnp.int32),
            pltpu.VMEM((K,), jnp.int32),
            pltpu.VMEM((K,), jnp.float32),
            pltpu.VMEM((K,), jnp.float32),
            pltpu.VMEM((K, D), jnp.float32),
            pltpu.VMEM((K, D), jnp.float32),
            pltpu.VMEM_SHARED((N, D), jnp.float32),
            pltpu.SemaphoreType.DMA,
            pltpu.SemaphoreType.DMA,
            pltpu.SemaphoreType.DMA,
        ],
        compiler_params=_sc_compiler_params(),
    )
    return f(ha, hb, src, dst, edge_weight, zeros)


def _dot(a, b):
    return lax.dot_general(a, b, (((1,), (0,)), ((), ())),
                           preferred_element_type=jnp.float32, precision=None)


def _tc1_kernel(x_ref, incp_ref, w0x_ref, w0e_ref, b0_ref,
                h0a_ref, h0b_ref, invd_ref):
    inc = incp_ref[0] + incp_ref[1]
    h = jnp.maximum(_dot(x_ref[...], w0x_ref[...])
                    + _dot(inc[:, :16], w0e_ref[...]) + b0_ref[...], 0.0)
    h0a_ref[...] = h[:, :HD]
    h0b_ref[...] = h[:, HD:]
    invd_ref[...] = 1.0 / jnp.maximum(inc[:, 16:17], 1.0)


def _tc1(x, incp, W0x, W0e, b0):
    grid = N // BLK
    return pl.pallas_call(
        _tc1_kernel,
        grid=(grid,),
        in_specs=[
            pl.BlockSpec((BLK, XD), lambda i: (i, 0)),
            pl.BlockSpec((2, BLK, 32), lambda i: (0, i, 0)),
            pl.BlockSpec((XD, P), lambda i: (0, 0)),
            pl.BlockSpec((16, P), lambda i: (0, 0)),
            pl.BlockSpec((1, P), lambda i: (0, 0)),
        ],
        out_specs=(
            pl.BlockSpec((BLK, HD), lambda i: (i, 0)),
            pl.BlockSpec((BLK, HD), lambda i: (i, 0)),
            pl.BlockSpec((BLK, 1), lambda i: (i, 0)),
        ),
        out_shape=(
            jax.ShapeDtypeStruct((N, HD), jnp.float32),
            jax.ShapeDtypeStruct((N, HD), jnp.float32),
            jax.ShapeDtypeStruct((N, 1), jnp.float32),
        ),
    )(x, incp, W0x, W0e, b0)


def _tc2_kernel(sa_ref, sb_ref, invd_ref, h0a_ref, h0b_ref, w_ref, b_ref,
                ha_ref, hb_ref):
    aggr = jnp.concatenate([sa_ref[...], sb_ref[...]], axis=1) * invd_ref[...]
    h0 = jnp.concatenate([h0a_ref[...], h0b_ref[...]], axis=1)
    h = jnp.maximum(h0 + _dot(aggr, w_ref[...]) + b_ref[...], 0.0)
    ha_ref[...] = h[:, :HD]
    hb_ref[...] = h[:, HD:]


def _tc2(sa, sb, invd, h0a, h0b, W, b):
    grid = N // BLK
    return pl.pallas_call(
        _tc2_kernel,
        grid=(grid,),
        in_specs=[
            pl.BlockSpec((BLK, HD), lambda i: (i, 0)),
            pl.BlockSpec((BLK, HD), lambda i: (i, 0)),
            pl.BlockSpec((BLK, 1), lambda i: (i, 0)),
            pl.BlockSpec((BLK, HD), lambda i: (i, 0)),
            pl.BlockSpec((BLK, HD), lambda i: (i, 0)),
            pl.BlockSpec((P, P), lambda i: (0, 0)),
            pl.BlockSpec((1, P), lambda i: (0, 0)),
        ],
        out_specs=(
            pl.BlockSpec((BLK, HD), lambda i: (i, 0)),
            pl.BlockSpec((BLK, HD), lambda i: (i, 0)),
        ),
        out_shape=(
            jax.ShapeDtypeStruct((N, HD), jnp.float32),
            jax.ShapeDtypeStruct((N, HD), jnp.float32),
        ),
    )(sa, sb, invd, h0a, h0b, W, b)


def _tc3_kernel(sa_ref, sb_ref, axa_ref, axb_ref, invd_ref, nw_ref, bt_ref,
                wfh_ref, wfx_ref, bf_ref, wm1_ref, bm1_ref, wm2_ref, bm2_ref,
                out_ref, acc):
    i = pl.program_id(0)
    n = pl.num_programs(0)

    @pl.when(i == 0)
    def _():
        acc[...] = jnp.zeros_like(acc)

    invd = invd_ref[...]
    aggh = jnp.concatenate([sa_ref[...], sb_ref[...]], axis=1) * invd
    aggx = jnp.concatenate([axa_ref[...], axb_ref[...]], axis=1) * invd
    h = jnp.maximum(_dot(aggh, wfh_ref[...]) + _dot(aggx, wfx_ref[...])
                    + bf_ref[...], 0.0)
    hw = h * nw_ref[...]
    bids = bt_ref[0, 0, :]
    onehot = (jax.lax.broadcasted_iota(jnp.int32, (NG, BLK), 0)
              == bids[None, :]).astype(jnp.float32)
    acc[:, :P] += _dot(onehot, hw)
    acc[:, P:] += jnp.sum(onehot, axis=1, keepdims=True)

    @pl.when(i == n - 1)
    def _():
        ge = acc[:, :P] / jnp.maximum(acc[:, P:], 1.0)
        hm = jnp.maximum(_dot(ge, wm1_ref[...]) + bm1_ref[...], 0.0)
        out_ref[...] = _dot(hm, wm2_ref[...]) + bm2_ref[...]


def _tc3(sa, sb, axa, axb, invd, nw, batch3, Wfh, Wfx, bf, Wm1, bm1, Wm2, bm2):
    grid = N // BLK
    return pl.pallas_call(
        _tc3_kernel,
        grid=(grid,),
        in_specs=[
            pl.BlockSpec((BLK, HD), lambda i: (i, 0)),
            pl.BlockSpec((BLK, HD), lambda i: (i, 0)),
            pl.BlockSpec((BLK, XH), lambda i: (i, 0)),
            pl.BlockSpec((BLK, XH), lambda i: (i, 0)),
            pl.BlockSpec((BLK, 1), lambda i: (i, 0)),
            pl.BlockSpec((BLK, 1), lambda i: (i, 0)),
            pl.BlockSpec((1, 1, BLK), lambda i: (i, 0, 0)),
            pl.BlockSpec((P, P), lambda i: (0, 0)),
            pl.BlockSpec((XD, P), lambda i: (0, 0)),
            pl.BlockSpec((1, P), lambda i: (0, 0)),
            pl.BlockSpec((P, P), lambda i: (0, 0)),
            pl.BlockSpec((1, P), lambda i: (0, 0)),
            pl.BlockSpec((P, 8), lambda i: (0, 0)),
            pl.BlockSpec((1, 8), lambda i: (0, 0)),
        ],
        out_specs=pl.BlockSpec((NG, 8), lambda i: (0, 0)),
        out_shape=jax.ShapeDtypeStruct((NG, 8), jnp.float32),
        scratch_shapes=[pltpu.VMEM((NG, P + 1), jnp.float32)],
    )(sa, sb, axa, axb, invd, nw, batch3, Wfh, Wfx, bf, Wm1, bm1, Wm2, bm2)


def _padw(W, r, c):
    return jnp.pad(W, ((0, r - W.shape[0]), (0, c - W.shape[1])))


def kernel(x, edge_index, edge_attr, edge_weight, node_weight, batch,
           W0, b0, W1, b1, W2, b2, W3, b3, Wf, bf, Wm1, bm1, Wm2, bm2):
    src = edge_index[0]
    dst = edge_index[1]

    W0x = _padw(W0[:XD], XD, P)
    W0e = _padw(W0[XD:], 16, P)
    b0p = jnp.pad(b0, (0, P - b0.shape[0])).reshape(1, P)
    W1p = _padw(W1, P, P)
    W2p = _padw(W2, P, P)
    W3p = _padw(W3, P, P)
    b1p = jnp.pad(b1, (0, P - b1.shape[0])).reshape(1, P)
    b2p = jnp.pad(b2, (0, P - b2.shape[0])).reshape(1, P)
    b3p = jnp.pad(b3, (0, P - b3.shape[0])).reshape(1, P)
    Wfh = _padw(Wf[:300], P, P)
    Wfx = _padw(Wf[300:], XD, P)
    bfp = jnp.pad(bf, (0, P - bf.shape[0])).reshape(1, P)
    Wm1p = _padw(Wm1, P, P)
    bm1p = jnp.pad(bm1, (0, P - bm1.shape[0])).reshape(1, P)
    Wm2p = _padw(Wm2, P, 8)
    bm2p = jnp.pad(bm2.reshape(1, 1), ((0, 0), (0, 7)))

    z32 = jnp.zeros((N, 32), jnp.float32)
    z64 = jnp.zeros((N, XH), jnp.float32)
    z160 = jnp.zeros((N, HD), jnp.float32)

    xa = x[:, :XH]
    xb = x[:, XH:]
    nw = node_weight.reshape(N, 1)
    batch3 = batch.reshape(N // BLK, 1, BLK)

    incp = _sc_inc_cnt(edge_attr, edge_weight, dst, z32)
    h0a, h0b, invd = _tc1(x, incp, W0x, W0e, b0p)

    axa, axb = _sc_gather_scatter(xa, xb, src, dst, edge_weight, z64, XH, 160)

    ha, hb = h0a, h0b
    for Wp, bp in ((W1p, b1p), (W2p, b2p), (W3p, b3p)):
        sa, sb = _sc_gather_scatter(ha, hb, src, dst, edge_weight, z160,
                                    HD, 80)
        ha, hb = _tc2(sa, sb, invd, h0a, h0b, Wp, bp)

    s4a, s4b = _sc_gather_scatter(ha, hb, src, dst, edge_weight, z160,
                                  HD, 80)
    out = _tc3(s4a, s4b, axa, axb, invd, nw, batch3,
               Wfh, Wfx, bfp, Wm1p, bm1p, Wm2p, bm2p)
    return out[:, 0]
